# baseline (speedup 1.0000x reference)
import functools

import jax
import jax.numpy as jnp
from jax import lax
from jax.experimental import pallas as pl
from jax.experimental.pallas import tpu as pltpu
from jax.experimental.pallas import tpu_sc as plsc

VOC_SIZE = 1000000
EMB_SIZE = 64
BATCH = 16384

NUM_CORES = 2
NUM_SUBCORES = 16
NUM_WORKERS = NUM_CORES * NUM_SUBCORES
BPW = BATCH // NUM_WORKERS
IDX_CHUNK = 128
NCHUNK = BPW // IDX_CHUNK
LANES = 16


def _rsqrt(v):
    xi = plsc.bitcast(v, jnp.int32)
    yi = jnp.full((LANES,), 0x5F3759DF, jnp.int32) - lax.shift_right_logical(
        xi, jnp.full((LANES,), 1, jnp.int32))
    y = plsc.bitcast(yi, jnp.float32)
    h = v * 0.5
    for _ in range(3):
        y = y * (1.5 - h * y * y)
    return y


def _body(x0_hbm, x1_hbm, table_hbm, out_hbm,
          idx0_v, idx1_v, rows0_v, rows1_v, out_v, sem):
    c = lax.axis_index("c")
    s = lax.axis_index("s")
    wid = s * NUM_CORES + c
    base = wid * BPW

    pltpu.sync_copy(x0_hbm.at[pl.ds(wid * NCHUNK, NCHUNK)], idx0_v)
    pltpu.sync_copy(x1_hbm.at[pl.ds(wid * NCHUNK, NCHUNK)], idx1_v)

    copies = []
    for k in range(NCHUNK):
        sl = pl.ds(k * IDX_CHUNK, IDX_CHUNK)
        copies.append(pltpu.async_copy(
            table_hbm.at[idx0_v.at[k]], rows0_v.at[sl], sem))
        copies.append(pltpu.async_copy(
            table_hbm.at[idx1_v.at[k]], rows1_v.at[sl], sem))
    for cp in copies:
        cp.wait()

    lanes = lax.iota(jnp.int32, LANES)
    zero = jnp.zeros((LANES,), jnp.float32)

    def group(g, carry):
        row_ids = lanes + g * LANES

        def col(j, acc):
            dot, n0, n1 = acc
            jv = jnp.full((LANES,), j, jnp.int32)
            a = plsc.load_gather(rows0_v, [row_ids, jv])
            b = plsc.load_gather(rows1_v, [row_ids, jv])
            return (dot + a * b, n0 + a * a, n1 + b * b)

        dot, n0, n1 = lax.fori_loop(0, EMB_SIZE, col, (zero, zero, zero),
                                    unroll=True)
        out_v[pl.ds(g * LANES, LANES)] = dot * _rsqrt(n0 * n1)
        return carry

    lax.fori_loop(0, BPW // LANES, group, 0)
    pltpu.sync_copy(out_v, out_hbm.at[pl.ds(base, BPW)])


@functools.partial(jax.jit, static_argnames=())
def _w2vec_sc(x0, x1, table):
    mesh = plsc.VectorSubcoreMesh(core_axis_name="c", subcore_axis_name="s")
    return pl.kernel(
        _body,
        mesh=mesh,
        out_type=jax.ShapeDtypeStruct((BATCH,), jnp.float32),
        scratch_types=[
            pltpu.VMEM((NCHUNK, IDX_CHUNK), jnp.int32),
            pltpu.VMEM((NCHUNK, IDX_CHUNK), jnp.int32),
            pltpu.VMEM((BPW, EMB_SIZE), jnp.float32),
            pltpu.VMEM((BPW, EMB_SIZE), jnp.float32),
            pltpu.VMEM((BPW,), jnp.float32),
            pltpu.SemaphoreType.DMA,
        ],
        compiler_params=pltpu.CompilerParams(
            needs_layout_passes=False, use_tc_tiling_on_sc=False),
    )(x0, x1, table)


def kernel(x, table):
    x0 = x[0].reshape(NUM_WORKERS * NCHUNK, IDX_CHUNK)
    x1 = x[1].reshape(NUM_WORKERS * NCHUNK, IDX_CHUNK)
    return _w2vec_sc(x0, x1, table)


# speedup vs baseline: 1.0017x; 1.0017x over previous
import jax
import jax.numpy as jnp
from jax import lax
from jax.experimental import pallas as pl
from jax.experimental.pallas import tpu as pltpu
from jax.experimental.pallas import tpu_sc as plsc

VOC_SIZE = 1000000
EMB_SIZE = 64
BATCH = 16384

NUM_CORES = 2
NUM_SUBCORES = 16
NUM_WORKERS = NUM_CORES * NUM_SUBCORES
BPW = BATCH // NUM_WORKERS
LANES = 16
NGROUP = BPW // LANES


def _rsqrt(v):
    xi = plsc.bitcast(v, jnp.int32)
    yi = jnp.full((LANES,), 0x5F3759DF, jnp.int32) - lax.shift_right_logical(
        xi, jnp.full((LANES,), 1, jnp.int32))
    y = plsc.bitcast(yi, jnp.float32)
    h = v * 0.5
    for _ in range(3):
        y = y * (1.5 - h * y * y)
    return y


def _body(x0_hbm, x1_hbm, table_hbm, out_hbm,
          ids0_v, ids1_v, rows0_v, rows1_v, out_v, sem0, sem1):
    c = lax.axis_index("c")
    s = lax.axis_index("s")
    wid = s * NUM_CORES + c
    base = wid * BPW

    pltpu.sync_copy(x0_hbm.at[pl.ds(base, BPW)], ids0_v)
    pltpu.sync_copy(x1_hbm.at[pl.ds(base, BPW)], ids1_v)

    cp0 = pltpu.async_copy(table_hbm.at[ids0_v], rows0_v, sem0)
    cp1 = pltpu.async_copy(table_hbm.at[ids1_v], rows1_v, sem1)
    cp0.wait()
    cp1.wait()

    iota = lax.iota(jnp.int32, LANES)
    zero = jnp.zeros((LANES,), jnp.float32)

    def group(g, carry):
        row_idx = g * LANES + iota
        dot, n0, n1 = zero, zero, zero
        for j in range(EMB_SIZE):
            cj = jnp.full((LANES,), j, jnp.int32)
            a = plsc.load_gather(rows0_v, [row_idx, cj])
            b = plsc.load_gather(rows1_v, [row_idx, cj])
            dot = dot + a * b
            n0 = n0 + a * a
            n1 = n1 + b * b
        out_v[pl.ds(g * LANES, LANES)] = dot * _rsqrt(n0 * n1)
        return carry

    lax.fori_loop(0, NGROUP, group, 0)
    pltpu.sync_copy(out_v, out_hbm.at[pl.ds(base, BPW)])


@jax.jit
def _w2vec_sc(x0, x1, table):
    mesh = plsc.VectorSubcoreMesh(core_axis_name="c", subcore_axis_name="s")
    return pl.kernel(
        _body,
        mesh=mesh,
        out_type=jax.ShapeDtypeStruct((BATCH,), jnp.float32),
        scratch_types=[
            pltpu.VMEM((BPW,), jnp.int32),
            pltpu.VMEM((BPW,), jnp.int32),
            pltpu.VMEM((BPW, EMB_SIZE), jnp.float32),
            pltpu.VMEM((BPW, EMB_SIZE), jnp.float32),
            pltpu.VMEM((BPW,), jnp.float32),
            pltpu.SemaphoreType.DMA,
            pltpu.SemaphoreType.DMA,
        ],
        compiler_params=pltpu.CompilerParams(
            needs_layout_passes=False, use_tc_tiling_on_sc=False),
    )(x0, x1, table)


def kernel(x, table):
    return _w2vec_sc(x[0], x[1], table)
